# baseline (speedup 1.0000x reference)
import functools

import jax
import jax.numpy as jnp
from jax.experimental import pallas as pl
from jax.experimental.pallas import tpu as pltpu

_REGIONS = [[0, 2], [1, 3], [4, 6, 8], [5, 7, 9], [10, 11, 12, 13],
            [14, 16, 18], [15, 17, 19], [20, 22, 24], [21, 23, 25],
            [26, 27, 28, 29, 30, 31]]
_F = 128
_N = 10
_H = 8
_C1 = 128
_C2 = 256


def _fwd(x_ref, w1_ref, b1_ref, m_ref, c1_ref, a1s_ref, c1s_ref, e40_ref,
         e10_ref, g1b_ref, w2g_ref, a2_ref, esum_ref, g2b_ref, cw1_ref,
         cb1_ref, cw2_ref, cb2_ref, o_ref):
    f32 = jnp.float32

    hw, a_src, a_dst = [], [], []
    off = 0
    for i, ch in enumerate(_REGIONS):
        k = len(ch) * _F
        xi = x_ref[:, off:off + k]
        h = jnp.dot(xi, w1_ref[off:off + k, :], preferred_element_type=f32)
        h = jnp.maximum(h + b1_ref[i:i + 1, :], 0.0)
        off += k
        hwi = jnp.dot(h, m_ref[i * _F:(i + 1) * _F, :],
                      preferred_element_type=f32) + c1_ref[i:i + 1, :]
        hw.append(hwi)
        sd = jnp.dot(h, a1s_ref[i * _F:(i + 1) * _F, :],
                     preferred_element_type=f32) + c1s_ref[i:i + 1, :]
        a_src.append(sd[:, 0:_H])
        a_dst.append(sd[:, _H:2 * _H])
    as_cat = jnp.concatenate(a_src, axis=1)

    _W = _N * _H
    ad_cat = jnp.concatenate(a_dst, axis=1)
    e = jnp.concatenate(
        [ad_cat + jnp.concatenate([as_cat[:, _H * j:_H * (j + 1)]] * _N,
                                  axis=1) for j in range(_N)], axis=1)
    e = jnp.where(e >= 0, e, 0.2 * e)
    m = functools.reduce(jnp.maximum,
                         [e[:, _W * j:_W * (j + 1)] for j in range(_N)])
    p = jnp.exp(e - jnp.concatenate([m] * _N, axis=1))
    den = functools.reduce(
        lambda a, b: a + b, [p[:, _W * j:_W * (j + 1)] for j in range(_N)])
    inv = 1.0 / den
    alphaf = p * jnp.concatenate([inv] * _N, axis=1)

    hw2 = []
    _G = 5
    for i in range(_N):
        u = None
        for jg in range(_N // _G):
            lhs = jnp.concatenate(
                [alphaf[:, _W * (_G * jg + r) + _H * i:
                        _W * (_G * jg + r) + _H * (i + 1)]
                 for r in range(_G)], axis=1)
            ajg = jnp.dot(lhs, e40_ref[...],
                          preferred_element_type=f32)
            for r in range(_G):
                t = ajg[:, _H * _C1 * r:_H * _C1 * (r + 1)] * hw[_G * jg + r]
                u = t if u is None else u + t
        u = u + g1b_ref[...]
        u = jnp.where(u > 0, u, jnp.exp(jnp.minimum(u, 0.0)) - 1.0)
        hw2.append(jnp.dot(u, w2g_ref[...], preferred_element_type=f32))

    sd2 = [jnp.dot(hw2[i], a2_ref[...], preferred_element_type=f32)
           for i in range(_N)]
    ss = jnp.concatenate([s[:, 0:1] for s in sd2], axis=1)
    dst = jnp.concatenate([s[:, 1:2] for s in sd2], axis=1)
    e2 = jnp.concatenate([dst] * _N, axis=1) \
        + jnp.dot(ss, e10_ref[...], preferred_element_type=f32)
    e2 = jnp.where(e2 >= 0, e2, 0.2 * e2)
    m2 = functools.reduce(jnp.maximum,
                          [e2[:, _N * j:_N * (j + 1)] for j in range(_N)])
    p2 = jnp.exp(e2 - jnp.concatenate([m2] * _N, axis=1))
    den2 = functools.reduce(
        lambda a, b: a + b, [p2[:, _N * j:_N * (j + 1)] for j in range(_N)])
    alpha2f = p2 * jnp.concatenate([1.0 / den2] * _N, axis=1)
    wmean = jnp.dot(alpha2f, esum_ref[...],
                    preferred_element_type=f32)
    pooled = None
    for j in range(_N):
        t = wmean[:, j:j + 1] * hw2[j]
        pooled = t if pooled is None else pooled + t
    pooled = pooled + g2b_ref[...]

    hc = jnp.dot(pooled, cw1_ref[...], preferred_element_type=f32)
    hc = jnp.maximum(hc + cb1_ref[...], 0.0)
    o_ref[...] = jnp.dot(hc, cw2_ref[...],
                         preferred_element_type=f32) + cb2_ref[...]


def kernel(x, enc_w1, enc_b1, enc_w2, enc_b2, wq, wk, gat1_w, gat1_att_src,
           gat1_att_dst, gat1_bias, gat2_w, gat2_att_src, gat2_att_dst,
           gat2_bias, cls_w1, cls_b1, cls_w2, cls_b2):
    b = x.shape[0]
    bb = min(512, b)
    perm = [c for ch in _REGIONS for c in ch]
    x2 = jnp.concatenate([x[:, c] for c in perm], axis=1)

    w1_all = jnp.concatenate([w.T for w in enc_w1], axis=0)
    b1_all = jnp.stack(enc_b1)
    eye_h = jnp.eye(_H, dtype=jnp.float32)
    s_src = jnp.einsum('hc,hk->hck', gat1_att_src, eye_h).reshape(_H * _C1, _H)
    s_dst = jnp.einsum('hc,hk->hck', gat1_att_dst, eye_h).reshape(_H * _C1, _H)
    a1 = jnp.concatenate([s_src, s_dst], axis=1)
    m_list = [(gat1_w @ w2).T for w2 in enc_w2]
    c1_list = [b2 @ gat1_w.T for b2 in enc_b2]
    m_all = jnp.concatenate(m_list, axis=0)
    c1_all = jnp.stack(c1_list)
    a1s_all = jnp.concatenate([mi @ a1 for mi in m_list], axis=0)
    c1s_all = jnp.stack([ci @ a1 for ci in c1_list])
    g1b = gat1_bias.reshape(1, _H * _C1)
    w2g = gat2_w.T
    a2 = jnp.concatenate([gat2_att_src.T, gat2_att_dst.T], axis=1)
    esum = jnp.kron(jnp.eye(_N, dtype=jnp.float32),
                    jnp.full((_N, 1), 1.0 / _N, jnp.float32))
    e40 = jnp.kron(jnp.eye(5 * _H, dtype=jnp.float32),
                   jnp.ones((1, _C1), jnp.float32))
    e10 = jnp.kron(jnp.eye(_N, dtype=jnp.float32),
                   jnp.ones((1, _N), jnp.float32))
    g2b = gat2_bias.reshape(1, _C2)
    cw1 = cls_w1.T
    cb1 = cls_b1.reshape(1, -1)
    cw2 = cls_w2.T
    cb2 = cls_b2.reshape(1, -1)

    def full(a):
        return pl.BlockSpec(a.shape, lambda i: (0,) * a.ndim)

    weights = (w1_all, b1_all, m_all, c1_all, a1s_all, c1s_all, e40, e10,
               g1b, w2g, a2, esum, g2b, cw1, cb1, cw2, cb2)
    out = pl.pallas_call(
        _fwd,
        grid=(b // bb,),
        in_specs=[pl.BlockSpec((bb, 32 * _F), lambda i: (i, 0))]
                 + [full(w) for w in weights],
        out_specs=pl.BlockSpec((bb, 2), lambda i: (i, 0)),
        out_shape=jax.ShapeDtypeStruct((b, 2), jnp.float32),
        compiler_params=pltpu.CompilerParams(
            dimension_semantics=("parallel",),
            vmem_limit_bytes=128 * 1024 * 1024),
    )(x2, *weights)
    return out


# speedup vs baseline: 1.0838x; 1.0838x over previous
import functools

import jax
import jax.numpy as jnp
from jax.experimental import pallas as pl
from jax.experimental.pallas import tpu as pltpu

_REGIONS = [[0, 2], [1, 3], [4, 6, 8], [5, 7, 9], [10, 11, 12, 13],
            [14, 16, 18], [15, 17, 19], [20, 22, 24], [21, 23, 25],
            [26, 27, 28, 29, 30, 31]]
_F = 128
_N = 10
_H = 8
_C1 = 128
_C2 = 256


def _fwd(x_ref, w1_ref, b1_ref, m_ref, c1_ref, a1s_ref, c1s_ref, e40_ref,
         e10_ref, g1b_ref, w2g_ref, a2_ref, esum_ref, g2b_ref, cw1_ref,
         cb1_ref, cw2_ref, cb2_ref, o_ref):
    f32 = jnp.float32

    hw, a_src, a_dst = [], [], []
    off = 0
    for i, ch in enumerate(_REGIONS):
        k = len(ch) * _F
        xi = x_ref[:, off:off + k]
        h = jnp.dot(xi, w1_ref[off:off + k, :], preferred_element_type=f32)
        h = jnp.maximum(h + b1_ref[i:i + 1, :], 0.0)
        off += k
        hwi = jnp.dot(h, m_ref[i * _F:(i + 1) * _F, :],
                      preferred_element_type=f32) + c1_ref[i:i + 1, :]
        hw.append(hwi)
        sd = jnp.dot(h, a1s_ref[i * _F:(i + 1) * _F, :],
                     preferred_element_type=f32) + c1s_ref[i:i + 1, :]
        a_src.append(sd[:, 0:_H])
        a_dst.append(sd[:, _H:2 * _H])
    as_cat = jnp.concatenate(a_src, axis=1)

    _W = _N * _H
    ad_cat = jnp.concatenate(a_dst, axis=1)
    e = jnp.concatenate(
        [ad_cat + jnp.concatenate([as_cat[:, _H * j:_H * (j + 1)]] * _N,
                                  axis=1) for j in range(_N)], axis=1)
    e = jnp.where(e >= 0, e, 0.2 * e)
    m = functools.reduce(jnp.maximum,
                         [e[:, _W * j:_W * (j + 1)] for j in range(_N)])
    p = jnp.exp(e - jnp.concatenate([m] * _N, axis=1))
    den = functools.reduce(
        lambda a, b: a + b, [p[:, _W * j:_W * (j + 1)] for j in range(_N)])
    inv = 1.0 / den
    alphaf = p * jnp.concatenate([inv] * _N, axis=1)

    hw2 = []
    _G = 5
    for i in range(_N):
        u = None
        for jg in range(_N // _G):
            lhs = jnp.concatenate(
                [alphaf[:, _W * (_G * jg + r) + _H * i:
                        _W * (_G * jg + r) + _H * (i + 1)]
                 for r in range(_G)], axis=1)
            ajg = jnp.dot(lhs, e40_ref[...],
                          preferred_element_type=f32)
            for r in range(_G):
                t = ajg[:, _H * _C1 * r:_H * _C1 * (r + 1)] * hw[_G * jg + r]
                u = t if u is None else u + t
        u = u + g1b_ref[...]
        u = jnp.where(u > 0, u, jnp.exp(jnp.minimum(u, 0.0)) - 1.0)
        hw2.append(jnp.dot(u, w2g_ref[...], preferred_element_type=f32))

    sd2 = [jnp.dot(hw2[i], a2_ref[...], preferred_element_type=f32)
           for i in range(_N)]
    ss = jnp.concatenate([s[:, 0:1] for s in sd2], axis=1)
    dst = jnp.concatenate([s[:, 1:2] for s in sd2], axis=1)
    e2 = jnp.concatenate([dst] * _N, axis=1) \
        + jnp.dot(ss, e10_ref[...], preferred_element_type=f32)
    e2 = jnp.where(e2 >= 0, e2, 0.2 * e2)
    m2 = functools.reduce(jnp.maximum,
                          [e2[:, _N * j:_N * (j + 1)] for j in range(_N)])
    p2 = jnp.exp(e2 - jnp.concatenate([m2] * _N, axis=1))
    den2 = functools.reduce(
        lambda a, b: a + b, [p2[:, _N * j:_N * (j + 1)] for j in range(_N)])
    alpha2f = p2 * jnp.concatenate([1.0 / den2] * _N, axis=1)
    wmean = jnp.dot(alpha2f, esum_ref[...],
                    preferred_element_type=f32)
    pooled = None
    for j in range(_N):
        t = wmean[:, j:j + 1] * hw2[j]
        pooled = t if pooled is None else pooled + t
    pooled = pooled + g2b_ref[...]

    hc = jnp.dot(pooled, cw1_ref[...], preferred_element_type=f32)
    hc = jnp.maximum(hc + cb1_ref[...], 0.0)
    o_ref[...] = jnp.dot(hc, cw2_ref[...],
                         preferred_element_type=f32) + cb2_ref[...]


def kernel(x, enc_w1, enc_b1, enc_w2, enc_b2, wq, wk, gat1_w, gat1_att_src,
           gat1_att_dst, gat1_bias, gat2_w, gat2_att_src, gat2_att_dst,
           gat2_bias, cls_w1, cls_b1, cls_w2, cls_b2):
    b = x.shape[0]
    bb = min(256, b)
    perm = [c for ch in _REGIONS for c in ch]
    x2 = jnp.concatenate([x[:, c] for c in perm], axis=1)

    w1_all = jnp.concatenate([w.T for w in enc_w1], axis=0)
    b1_all = jnp.stack(enc_b1)
    eye_h = jnp.eye(_H, dtype=jnp.float32)
    s_src = jnp.einsum('hc,hk->hck', gat1_att_src, eye_h).reshape(_H * _C1, _H)
    s_dst = jnp.einsum('hc,hk->hck', gat1_att_dst, eye_h).reshape(_H * _C1, _H)
    a1 = jnp.concatenate([s_src, s_dst], axis=1)
    m_list = [(gat1_w @ w2).T for w2 in enc_w2]
    c1_list = [b2 @ gat1_w.T for b2 in enc_b2]
    m_all = jnp.concatenate(m_list, axis=0)
    c1_all = jnp.stack(c1_list)
    a1s_all = jnp.concatenate([mi @ a1 for mi in m_list], axis=0)
    c1s_all = jnp.stack([ci @ a1 for ci in c1_list])
    g1b = gat1_bias.reshape(1, _H * _C1)
    w2g = gat2_w.T
    a2 = jnp.concatenate([gat2_att_src.T, gat2_att_dst.T], axis=1)
    esum = jnp.kron(jnp.eye(_N, dtype=jnp.float32),
                    jnp.full((_N, 1), 1.0 / _N, jnp.float32))
    e40 = jnp.kron(jnp.eye(5 * _H, dtype=jnp.float32),
                   jnp.ones((1, _C1), jnp.float32))
    e10 = jnp.kron(jnp.eye(_N, dtype=jnp.float32),
                   jnp.ones((1, _N), jnp.float32))
    g2b = gat2_bias.reshape(1, _C2)
    cw1 = cls_w1.T
    cb1 = cls_b1.reshape(1, -1)
    cw2 = cls_w2.T
    cb2 = cls_b2.reshape(1, -1)

    def full(a):
        return pl.BlockSpec(a.shape, lambda i: (0,) * a.ndim)

    weights = (w1_all, b1_all, m_all, c1_all, a1s_all, c1s_all, e40, e10,
               g1b, w2g, a2, esum, g2b, cw1, cb1, cw2, cb2)
    out = pl.pallas_call(
        _fwd,
        grid=(b // bb,),
        in_specs=[pl.BlockSpec((bb, 32 * _F), lambda i: (i, 0))]
                 + [full(w) for w in weights],
        out_specs=pl.BlockSpec((bb, 2), lambda i: (i, 0)),
        out_shape=jax.ShapeDtypeStruct((b, 2), jnp.float32),
        compiler_params=pltpu.CompilerParams(
            dimension_semantics=("parallel",),
            vmem_limit_bytes=128 * 1024 * 1024),
    )(x2, *weights)
    return out
